# baseline (speedup 1.0000x reference)
import functools

import jax
import jax.numpy as jnp
from jax import lax
from jax.experimental import pallas as pl
from jax.experimental.pallas import tpu as pltpu
from jax.experimental.pallas import tpu_sc as plsc

B = 16384
D = 64
NC = 2
NS = 16
NW = NC * NS
BPW = B // NW
CHUNK = 128
NCHUNK = BPW // CHUNK
L = 16


def _body(user_hbm, item_hbm, ut_hbm, it_hbm, bu_hbm, bi_hbm, gb_hbm,
          out_hbm,
          uidx_v, iidx_v, urows_v, irows_v, ubias_v, ibias_v, gb_v, out_v,
          sem):
    wid = lax.axis_index("s") * NC + lax.axis_index("c")

    pltpu.sync_copy(user_hbm.at[wid], uidx_v)
    pltpu.sync_copy(item_hbm.at[wid], iidx_v)
    pltpu.sync_copy(gb_hbm, gb_v)

    copies = []
    for j in range(NCHUNK):
        sl = pl.ds(j * CHUNK, CHUNK)
        copies.append(pltpu.async_copy(ut_hbm.at[uidx_v.at[j]],
                                       urows_v.at[sl], sem))
        copies.append(pltpu.async_copy(it_hbm.at[iidx_v.at[j]],
                                       irows_v.at[sl], sem))
        copies.append(pltpu.async_copy(bu_hbm.at[uidx_v.at[j]],
                                       ubias_v.at[sl], sem))
        copies.append(pltpu.async_copy(bi_hbm.at[iidx_v.at[j]],
                                       ibias_v.at[sl], sem))
    for c in copies:
        c.wait()

    gbv = gb_v[...]
    lane = lax.iota(jnp.int32, 16)

    dnums = lax.GatherDimensionNumbers(
        offset_dims=(), collapsed_slice_dims=(0,), start_index_map=(0,))

    def shuffle(x, idx):
        return lax.gather(x, idx[:, None], dnums, (1,),
                          mode=lax.GatherScatterMode.PROMISE_IN_BOUNDS)

    def lanesum(p):
        for sh in (8, 4, 2, 1):
            p = p + shuffle(p, lane ^ sh)
        return p

    def group(g, carry):
        base = g * L
        acc = jnp.zeros((L,), jnp.float32)
        for r in range(L):
            b = base + r
            p = (urows_v[b, pl.ds(0, 16)] * irows_v[b, pl.ds(0, 16)]
                 + urows_v[b, pl.ds(16, 16)] * irows_v[b, pl.ds(16, 16)]
                 + urows_v[b, pl.ds(32, 16)] * irows_v[b, pl.ds(32, 16)]
                 + urows_v[b, pl.ds(48, 16)] * irows_v[b, pl.ds(48, 16)])
            acc = jnp.where(lane == r, lanesum(p), acc)
        res = acc + gbv + ubias_v[pl.ds(base, L)] + ibias_v[pl.ds(base, L)]
        out_v[pl.ds(base, L)] = jnp.minimum(jnp.maximum(res, 1.0), 5.0)
        return carry

    lax.fori_loop(0, BPW // L, group, 0)

    pltpu.sync_copy(out_v, out_hbm.at[pl.ds(wid * BPW, BPW)])


@jax.jit
def _svd_score(user, item, user_table, item_table, bias_user_flat,
               bias_item_flat, gb16):
    mesh = plsc.VectorSubcoreMesh(core_axis_name="c", subcore_axis_name="s")
    k = functools.partial(
        pl.kernel,
        out_type=jax.ShapeDtypeStruct((B,), jnp.float32),
        mesh=mesh,
        scratch_types=[
            pltpu.VMEM((NCHUNK, CHUNK), jnp.int32),
            pltpu.VMEM((NCHUNK, CHUNK), jnp.int32),
            pltpu.VMEM((BPW, D), jnp.float32),
            pltpu.VMEM((BPW, D), jnp.float32),
            pltpu.VMEM((BPW,), jnp.float32),
            pltpu.VMEM((BPW,), jnp.float32),
            pltpu.VMEM((16,), jnp.float32),
            pltpu.VMEM((BPW,), jnp.float32),
            pltpu.SemaphoreType.DMA,
        ],
        compiler_params=pltpu.CompilerParams(use_tc_tiling_on_sc=False),
    )(_body)
    user_r = user.reshape(NW, NCHUNK, CHUNK)
    item_r = item.reshape(NW, NCHUNK, CHUNK)
    return k(user_r, item_r, user_table, item_table,
             bias_user_flat, bias_item_flat, gb16)


def kernel(user, item, user_table, item_table, bias_user_table,
           bias_item_table, global_bias):
    gb16 = jnp.broadcast_to(
        jnp.asarray(global_bias, jnp.float32).reshape(1), (16,))
    out = _svd_score(user, item, user_table, item_table,
                     bias_user_table.reshape(-1), bias_item_table.reshape(-1),
                     gb16)
    return out.reshape(1, B)
